# baseline (speedup 1.0000x reference)
import jax
import jax.numpy as jnp
from jax import lax
from jax.experimental import pallas as pl
from jax.experimental.pallas import tpu as pltpu

_B = 16
_C = 3
_H = 384
_W = 384
_BB = 4
_EPS2 = 1e-16


def _body(o_ref, t_ref, out_ref, acc_ref, cnt_ref):
    i = pl.program_id(0)

    @pl.when(i == 0)
    def _init():
        acc_ref[...] = jnp.zeros_like(acc_ref)
        cnt_ref[...] = jnp.zeros_like(cnt_ref)

    o = o_ref[...]
    t = t_ref[...]
    dot = jnp.sum(o * t, axis=1)
    no2 = jnp.sum(o * o, axis=1)
    nt2 = jnp.sum(t * t, axis=1)
    r = lax.rsqrt(jnp.maximum(no2, _EPS2) * jnp.maximum(nt2, _EPS2))
    cos = dot * r
    cnt_v = jnp.where(nt2 > 0.0, 1.0, 0.0)
    acc_ref[...] += jnp.sum(cos.reshape(_BB * _H // 8, 8, _W), axis=0)
    cnt_ref[...] += jnp.sum(cnt_v.reshape(_BB * _H // 8, 8, _W), axis=0)

    @pl.when(i == pl.num_programs(0) - 1)
    def _fin():
        cnt = jnp.sum(cnt_ref[...])
        loss = (cnt - jnp.sum(acc_ref[...])) / cnt
        out_ref[...] = loss.reshape(1, 1)


def kernel(output, target):
    out = pl.pallas_call(
        _body,
        grid=(_B // _BB,),
        in_specs=[
            pl.BlockSpec((_BB, _C, _H, _W), lambda i: (i, 0, 0, 0)),
            pl.BlockSpec((_BB, _C, _H, _W), lambda i: (i, 0, 0, 0)),
        ],
        out_specs=pl.BlockSpec((1, 1), lambda i: (0, 0)),
        out_shape=jax.ShapeDtypeStruct((1, 1), jnp.float32),
        scratch_shapes=[
            pltpu.VMEM((8, _W), jnp.float32),
            pltpu.VMEM((8, _W), jnp.float32),
        ],
    )(output, target)
    return out[0, 0]


# speedup vs baseline: 1.0170x; 1.0170x over previous
import jax
import jax.numpy as jnp
from jax import lax
from jax.experimental import pallas as pl
from jax.experimental.pallas import tpu as pltpu

_B = 16
_C = 3
_H = 384
_W = 384
_BB = 2
_EPS2 = 1e-16


def _body(o_ref, t_ref, out_ref, acc_ref, cnt_ref):
    i = pl.program_id(0)

    @pl.when(i == 0)
    def _init():
        acc_ref[...] = jnp.zeros_like(acc_ref)
        cnt_ref[...] = jnp.zeros_like(cnt_ref)

    o = o_ref[...]
    t = t_ref[...]
    dot = jnp.sum(o * t, axis=1)
    no2 = jnp.sum(o * o, axis=1)
    nt2 = jnp.sum(t * t, axis=1)
    r = lax.rsqrt(jnp.maximum(no2, _EPS2) * jnp.maximum(nt2, _EPS2))
    cos = dot * r
    cnt_v = jnp.where(nt2 > 0.0, 1.0, 0.0)
    acc_ref[...] += jnp.sum(cos.reshape(_BB * _H // 8, 8, _W), axis=0)
    cnt_ref[...] += jnp.sum(cnt_v.reshape(_BB * _H // 8, 8, _W), axis=0)

    @pl.when(i == pl.num_programs(0) - 1)
    def _fin():
        cnt = jnp.sum(cnt_ref[...])
        loss = (cnt - jnp.sum(acc_ref[...])) / cnt
        out_ref[...] = loss.reshape(1, 1)


def kernel(output, target):
    out = pl.pallas_call(
        _body,
        grid=(_B // _BB,),
        in_specs=[
            pl.BlockSpec((_BB, _C, _H, _W), lambda i: (i, 0, 0, 0)),
            pl.BlockSpec((_BB, _C, _H, _W), lambda i: (i, 0, 0, 0)),
        ],
        out_specs=pl.BlockSpec((1, 1), lambda i: (0, 0)),
        out_shape=jax.ShapeDtypeStruct((1, 1), jnp.float32),
        scratch_shapes=[
            pltpu.VMEM((8, _W), jnp.float32),
            pltpu.VMEM((8, _W), jnp.float32),
        ],
    )(output, target)
    return out[0, 0]


# speedup vs baseline: 1.0295x; 1.0123x over previous
import jax
import jax.numpy as jnp
from jax import lax
from jax.experimental import pallas as pl
from jax.experimental.pallas import tpu as pltpu

_B = 16
_C = 3
_H = 384
_W = 384
_EPS2 = 1e-16


def _fold(o, t):
    dot = jnp.sum(o * t, axis=1)
    no2 = jnp.sum(o * o, axis=1)
    nt2 = jnp.sum(t * t, axis=1)
    r = lax.rsqrt(jnp.maximum(no2, _EPS2) * jnp.maximum(nt2, _EPS2))
    cos = dot * r
    cnt_v = jnp.where(nt2 > 0.0, 1.0, 0.0)
    return (jnp.sum(cos.reshape(_H // 8, 8, _W), axis=0),
            jnp.sum(cnt_v.reshape(_H // 8, 8, _W), axis=0))


def _body(oa_ref, ob_ref, ta_ref, tb_ref, out_ref, acc_ref, cnt_ref):
    i = pl.program_id(0)

    @pl.when(i == 0)
    def _init():
        acc_ref[...] = jnp.zeros_like(acc_ref)
        cnt_ref[...] = jnp.zeros_like(cnt_ref)

    ca, na = _fold(oa_ref[...], ta_ref[...])
    cb, nb = _fold(ob_ref[...], tb_ref[...])
    acc_ref[...] += ca + cb
    cnt_ref[...] += na + nb

    @pl.when(i == pl.num_programs(0) - 1)
    def _fin():
        cnt = jnp.sum(cnt_ref[...])
        loss = (cnt - jnp.sum(acc_ref[...])) / cnt
        out_ref[...] = loss.reshape(1, 1)


def kernel(output, target):
    spec_a = pl.BlockSpec((1, _C, _H, _W), lambda i: (2 * i, 0, 0, 0))
    spec_b = pl.BlockSpec((1, _C, _H, _W), lambda i: (2 * i + 1, 0, 0, 0))
    out = pl.pallas_call(
        _body,
        grid=(_B // 2,),
        in_specs=[spec_a, spec_b, spec_a, spec_b],
        out_specs=pl.BlockSpec((1, 1), lambda i: (0, 0)),
        out_shape=jax.ShapeDtypeStruct((1, 1), jnp.float32),
        scratch_shapes=[
            pltpu.VMEM((8, _W), jnp.float32),
            pltpu.VMEM((8, _W), jnp.float32),
        ],
    )(output, output, target, target)
    return out[0, 0]


# speedup vs baseline: 1.0515x; 1.0213x over previous
import jax
import jax.numpy as jnp
from jax import lax
from jax.experimental import pallas as pl
from jax.experimental.pallas import tpu as pltpu

_B = 16
_C = 3
_H = 384
_W = 384
_EPS2 = 1e-16


def _fold(o, t):
    dot = jnp.sum(o * t, axis=1)
    no2 = jnp.sum(o * o, axis=1)
    nt2 = jnp.sum(t * t, axis=1)
    r = lax.rsqrt(jnp.maximum(no2, _EPS2) * jnp.maximum(nt2, _EPS2))
    cos = dot * r
    cnt_v = jnp.where(nt2 > 0.0, 1.0, 0.0)
    return (jnp.sum(cos.reshape(_H // 8, 8, _W), axis=0),
            jnp.sum(cnt_v.reshape(_H // 8, 8, _W), axis=0))


def _body(oa_ref, ob_ref, oc_ref, od_ref, ta_ref, tb_ref, tc_ref, td_ref,
          out_ref, acc_ref, cnt_ref):
    i = pl.program_id(0)

    @pl.when(i == 0)
    def _init():
        acc_ref[...] = jnp.zeros_like(acc_ref)
        cnt_ref[...] = jnp.zeros_like(cnt_ref)

    ca, na = _fold(oa_ref[...], ta_ref[...])
    cb, nb = _fold(ob_ref[...], tb_ref[...])
    cc, nc = _fold(oc_ref[...], tc_ref[...])
    cd, nd = _fold(od_ref[...], td_ref[...])
    acc_ref[...] += (ca + cb) + (cc + cd)
    cnt_ref[...] += (na + nb) + (nc + nd)

    @pl.when(i == pl.num_programs(0) - 1)
    def _fin():
        cnt = jnp.sum(cnt_ref[...])
        loss = (cnt - jnp.sum(acc_ref[...])) / cnt
        out_ref[...] = loss.reshape(1, 1)


def kernel(output, target):
    specs = [pl.BlockSpec((1, _C, _H, _W),
                          (lambda k: (lambda i: (4 * i + k, 0, 0, 0)))(k))
             for k in range(4)]
    out = pl.pallas_call(
        _body,
        grid=(_B // 4,),
        in_specs=specs + specs,
        out_specs=pl.BlockSpec((1, 1), lambda i: (0, 0)),
        out_shape=jax.ShapeDtypeStruct((1, 1), jnp.float32),
        scratch_shapes=[
            pltpu.VMEM((8, _W), jnp.float32),
            pltpu.VMEM((8, _W), jnp.float32),
        ],
    )(output, output, output, output, target, target, target, target)
    return out[0, 0]


# speedup vs baseline: 1.0571x; 1.0054x over previous
import jax
import jax.numpy as jnp
from jax import lax
from jax.experimental import pallas as pl
from jax.experimental.pallas import tpu as pltpu

_B = 16
_C = 3
_H = 384
_W = 384
_HH = _H // 2
_EPS2 = 1e-16


def _fold(o, t):
    dot = jnp.sum(o * t, axis=1)
    no2 = jnp.sum(o * o, axis=1)
    nt2 = jnp.sum(t * t, axis=1)
    r = lax.rsqrt(jnp.maximum(no2, _EPS2) * jnp.maximum(nt2, _EPS2))
    cos = dot * r
    cnt_v = jnp.where(nt2 > 0.0, 1.0, 0.0)
    return (jnp.sum(cos.reshape(_HH // 8, 8, _W), axis=0),
            jnp.sum(cnt_v.reshape(_HH // 8, 8, _W), axis=0))


def _body(o0, o1, o2, o3, o4, o5, o6, o7,
          t0, t1, t2, t3, t4, t5, t6, t7, out_ref, acc_ref, cnt_ref):
    i = pl.program_id(0)

    @pl.when(i == 0)
    def _init():
        acc_ref[...] = jnp.zeros_like(acc_ref)
        cnt_ref[...] = jnp.zeros_like(cnt_ref)

    os_ = (o0, o1, o2, o3, o4, o5, o6, o7)
    ts_ = (t0, t1, t2, t3, t4, t5, t6, t7)
    accs = []
    cnts = []
    for orf, trf in zip(os_, ts_):
        a, c = _fold(orf[...], trf[...])
        accs.append(a)
        cnts.append(c)
    acc_ref[...] += sum(accs[1:], accs[0])
    cnt_ref[...] += sum(cnts[1:], cnts[0])

    @pl.when(i == pl.num_programs(0) - 1)
    def _fin():
        cnt = jnp.sum(cnt_ref[...])
        loss = (cnt - jnp.sum(acc_ref[...])) / cnt
        out_ref[...] = loss.reshape(1, 1)


def kernel(output, target):
    specs = [pl.BlockSpec((1, _C, _HH, _W),
                          (lambda k: (lambda i: (4 * i + k // 2, 0, k % 2, 0)))(k))
             for k in range(8)]
    out = pl.pallas_call(
        _body,
        grid=(_B // 4,),
        in_specs=specs + specs,
        out_specs=pl.BlockSpec((1, 1), lambda i: (0, 0)),
        out_shape=jax.ShapeDtypeStruct((1, 1), jnp.float32),
        scratch_shapes=[
            pltpu.VMEM((8, _W), jnp.float32),
            pltpu.VMEM((8, _W), jnp.float32),
        ],
    )(*([output] * 8), *([target] * 8))
    return out[0, 0]
